# Initial kernel scaffold; baseline (speedup 1.0000x reference)
import jax
import jax.numpy as jnp
from jax.experimental import pallas as pl


def kernel(x, edge_index, edge_type, visual, textual, struct_emb, rel_emb, W1_s, b1_s, W2_s, W1_v, b1_v, W2_v, W1_t, b1_t, W2_t, Wv_proj, bv_proj, Wt_proj, bt_proj, alpha, gamma):
    raise NotImplementedError("write your pallas kernel here")


# speedup vs baseline: 6.9941x; 6.9941x over previous
import functools
import jax
import jax.numpy as jnp
from jax import lax
from jax.experimental import pallas as pl
from jax.experimental.pallas import tpu as pltpu
from jax.experimental.pallas import tpu_sc as plsc

N = 10000
NPAD = 10240
E = 320000
D = 128
NREL = 200
NW = 32
EPT = E // NW
BT = 80
NBATCH = EPT // BT
NB = 1000
SROWS = 80


def _leaky_j(v):
    return jnp.where(v >= 0, v, 0.01 * v)


def _dense_body(vis, txt, st, rel,
                Wv, bv, Wt, bt,
                W1s, b1s, W2s, W1v, b1v, W2v, W1t, b1t, W2t,
                ap_s, b_s, p_s, q_s, c_s, r_s,
                ap_v, b_v, p_v, q_v, c_v, r_v,
                ap_t, b_t, p_t, q_t, c_t, r_t,
                mx):
    i = pl.program_id(0)
    f32 = jnp.float32

    def mm_t(a, w):
        return lax.dot_general(a, w, (((1,), (1,)), ((), ())),
                               preferred_element_type=f32)

    fv = mm_t(vis[...], Wv[...]) + bv[...]
    ft = mm_t(txt[...], Wt[...]) + bt[...]
    fs = st[...]

    scal = []
    for feat, W1, b1, W2, apo, bo, po, qo, co, ro in (
            (fs, W1s, b1s, W2s, ap_s, b_s, p_s, q_s, c_s, r_s),
            (fv, W1v, b1v, W2v, ap_v, b_v, p_v, q_v, c_v, r_v),
            (ft, W1t, b1t, W2t, ap_t, b_t, p_t, q_t, c_t, r_t)):
        W1m = W1[...]
        Ap = mm_t(feat, W1m[:, :D]) + b1[...]
        B = mm_t(feat, W1m[:, D:2 * D])
        C = mm_t(rel[...], W1m[:, 2 * D:])
        p2 = mm_t(Ap, W2[...])
        q2 = mm_t(B, W2[...])
        r2 = mm_t(C, W2[...])
        apo[...] = Ap
        bo[...] = B
        po[...] = p2
        qo[...] = q2
        co[...] = C
        ro[...] = r2
        scal.append((jnp.max(p2), jnp.max(q2), jnp.max(r2)))

    rr = lax.broadcasted_iota(jnp.int32, (8, 128), 0)
    cc = lax.broadcasted_iota(jnp.int32, (8, 128), 1)
    vals = jnp.full((8, 128), -1e30, f32)
    for mi, (pm, qm, rm) in enumerate(scal):
        vals = jnp.where((rr == 0) & (cc == mi), pm, vals)
        vals = jnp.where((rr == 1) & (cc == mi), qm, vals)
        vals = jnp.where((rr == 2) & (cc == mi), rm, vals)
    prev = jnp.where(i == 0, jnp.full((8, 128), -1e30, f32), mx[...])
    mx[...] = jnp.maximum(prev, vals)


def _run_dense(vis, txt, st, rel, Wv, bv, Wt, bt,
               W1s, b1s, W2s, W1v, b1v, W2v, W1t, b1t, W2t):
    f32 = jnp.float32
    grid = (N // NB,)
    nb = lambda i: (i, 0)
    z2 = lambda i: (0, 0)
    node2 = lambda shp: pl.BlockSpec((NB, shp), nb)
    full2 = lambda a, b: pl.BlockSpec((a, b), z2)
    in_specs = [
        node2(2048), node2(768), node2(D), full2(NREL, 64),
        full2(D, 2048), full2(1, D), full2(D, 768), full2(1, D),
        full2(D, 2 * D + 64), full2(1, D), full2(1, D),
        full2(D, 2 * D + 64), full2(1, D), full2(1, D),
        full2(D, 2 * D + 64), full2(1, D), full2(1, D),
    ]
    per_mod_out = [
        jax.ShapeDtypeStruct((N, D), f32),
        jax.ShapeDtypeStruct((N, D), f32),
        jax.ShapeDtypeStruct((N, 1), f32),
        jax.ShapeDtypeStruct((N, 1), f32),
        jax.ShapeDtypeStruct((NREL, D), f32),
        jax.ShapeDtypeStruct((NREL, 1), f32),
    ]
    per_mod_spec = [
        node2(D), node2(D),
        pl.BlockSpec((NB, 1), nb), pl.BlockSpec((NB, 1), nb),
        full2(NREL, D), full2(NREL, 1),
    ]
    out_shapes = per_mod_out * 3 + [jax.ShapeDtypeStruct((8, 128), f32)]
    out_specs = per_mod_spec * 3 + [full2(8, 128)]
    return pl.pallas_call(
        _dense_body, grid=grid, in_specs=in_specs,
        out_specs=out_specs, out_shape=out_shapes,
    )(vis, txt, st, rel, Wv, bv, Wt, bt,
      W1s, b1s, W2s, W1v, b1v, W2v, W1t, b1t, W2t)


def _passa_body(dst_h, src_h, et_h, pq_h, r_h, m_h, zz_h,
                s_out, ev0, ev1, ev2,
                pq_v, r_v, m_v, dbuf, sbuf, tbuf, ebuf, ibuf,
                s_sh, sem):
    evs = (ev0, ev1, ev2)
    cid = lax.axis_index("c")
    sid = lax.axis_index("s")
    wid = sid * 2 + cid

    pltpu.sync_copy(pq_h, pq_v)
    pltpu.sync_copy(r_h, r_v)
    pltpu.sync_copy(m_h, m_v)

    @pl.when(sid == 0)
    def _():
        pltpu.sync_copy(zz_h, s_sh)

    plsc.subcore_barrier()

    base = wid * EPT
    mvreg = m_v[pl.ds(0, 16)]

    def batch(b, _):
        off = base + b * BT
        pltpu.sync_copy(dst_h.at[pl.ds(off, BT)], dbuf)
        pltpu.sync_copy(src_h.at[pl.ds(off, BT)], sbuf)
        pltpu.sync_copy(et_h.at[pl.ds(off, BT)], tbuf)
        for m in range(3):
            Mm = mvreg[m]
            for g in range(BT // 16):
                sl = pl.ds(g * 16, 16)
                dv = dbuf[sl]
                sv = sbuf[sl]
                tv = tbuf[sl]
                pg = plsc.load_gather(pq_v, [dv + (2 * m) * N])
                qg = plsc.load_gather(pq_v, [sv + (2 * m + 1) * N])
                rg = plsc.load_gather(r_v, [tv + m * NREL])
                v = pg + qg + rg
                lg = jnp.where(v >= 0, v, 0.01 * v)
                e = jnp.exp(lg - Mm)
                ebuf[m, sl] = e
                ibuf[sl] = dv + m * NPAD
            pltpu.sync_copy(ebuf.at[m], s_sh.at[ibuf], add=True)
        for m in range(3):
            pltpu.sync_copy(ebuf.at[m], evs[m].at[pl.ds(off, BT)])
        return 0

    lax.fori_loop(0, NBATCH, batch, 0)
    plsc.subcore_barrier()

    @pl.when(sid == 0)
    def _():
        pltpu.sync_copy(s_sh, s_out.at[cid])


def _run_passa(dst, src, et, pq, rtab, mvec, zz):
    f32 = jnp.float32
    mesh = plsc.VectorSubcoreMesh(core_axis_name="c", subcore_axis_name="s")
    out_type = (
        jax.ShapeDtypeStruct((2, 3 * NPAD), f32),
        jax.ShapeDtypeStruct((E,), f32),
        jax.ShapeDtypeStruct((E,), f32),
        jax.ShapeDtypeStruct((E,), f32),
    )
    scratch = [
        pltpu.VMEM((6 * N,), f32),
        pltpu.VMEM((3 * NREL,), f32),
        pltpu.VMEM((16,), f32),
        pltpu.VMEM((BT,), jnp.int32),
        pltpu.VMEM((BT,), jnp.int32),
        pltpu.VMEM((BT,), jnp.int32),
        pltpu.VMEM((3, BT), f32),
        pltpu.VMEM((BT,), jnp.int32),
        pltpu.VMEM_SHARED((3 * NPAD,), f32),
        pltpu.SemaphoreType.DMA,
    ]
    fn = pl.kernel(_passa_body, out_type, mesh=mesh, scratch_types=scratch,
                   compiler_params=pltpu.CompilerParams(
                       needs_layout_passes=False))
    return fn(dst, src, et, pq, rtab, mvec, zz)


def _passb_body(dst_h, src_h, et_h, ev_h, s0_h, s1_h, b_h, c_h, zz_h,
                agg_out,
                stot, sb2, dbuf, sbuf, tbuf, vbuf, Bbuf, Cbuf,
                agg_sh, sem):
    cid = lax.axis_index("c")
    sid = lax.axis_index("s")
    wid = sid * 2 + cid

    pltpu.sync_copy(s0_h, stot)
    pltpu.sync_copy(s1_h, sb2)

    def sadd(j, _):
        sl = pl.ds(j * 16, 16)
        stot[sl] = stot[sl] + sb2[sl]
        return 0

    lax.fori_loop(0, NPAD // 16, sadd, 0)

    rbase = sid * (NPAD // 16)
    pltpu.sync_copy(zz_h.at[pl.ds(rbase, NPAD // 16)],
                    agg_sh.at[pl.ds(rbase, NPAD // 16)])

    plsc.subcore_barrier()

    base = wid * EPT

    def batch(b, _):
        off = base + b * BT
        pltpu.sync_copy(dst_h.at[pl.ds(off, BT)], dbuf)
        pltpu.sync_copy(src_h.at[pl.ds(off, BT)], sbuf)
        pltpu.sync_copy(et_h.at[pl.ds(off, BT)], tbuf)
        pltpu.sync_copy(ev_h.at[pl.ds(off, BT)], vbuf)
        cp1 = pltpu.async_copy(b_h.at[sbuf], Bbuf, sem)
        cp1.wait()
        cp2 = pltpu.async_copy(c_h.at[tbuf], Cbuf, sem)
        cp2.wait()
        for g in range(BT // 16):
            sl = pl.ds(g * 16, 16)
            dv = dbuf[sl]
            sg = plsc.load_gather(stot, [dv])
            av = vbuf[sl] / sg
            for l in range(16):
                a = av[l]
                j = g * 16 + l
                for f in range(8):
                    fsl = pl.ds(f * 16, 16)
                    Bbuf[j, fsl] = (Bbuf[j, fsl] + Cbuf[j, fsl]) * a
        pltpu.sync_copy(Bbuf, agg_sh.at[dbuf], add=True)
        return 0

    lax.fori_loop(0, NBATCH, batch, 0)
    plsc.subcore_barrier()

    @pl.when(sid == 0)
    def _():
        pltpu.sync_copy(agg_sh, agg_out.at[cid])


def _run_passb(dst, src, et, ev_m, s0f, s1f, Bt, Ct, zagg):
    f32 = jnp.float32
    mesh = plsc.VectorSubcoreMesh(core_axis_name="c", subcore_axis_name="s")
    out_type = jax.ShapeDtypeStruct((2, NPAD, 128), f32)
    scratch = [
        pltpu.VMEM((NPAD,), f32),
        pltpu.VMEM((NPAD,), f32),
        pltpu.VMEM((BT,), jnp.int32),
        pltpu.VMEM((BT,), jnp.int32),
        pltpu.VMEM((BT,), jnp.int32),
        pltpu.VMEM((BT,), f32),
        pltpu.VMEM((BT, 128), f32),
        pltpu.VMEM((BT, 128), f32),
        pltpu.VMEM_SHARED((NPAD, 128), f32),
        pltpu.SemaphoreType.DMA,
    ]
    fn = pl.kernel(_passb_body, out_type, mesh=mesh, scratch_types=scratch,
                   compiler_params=pltpu.CompilerParams(
                       needs_layout_passes=False))
    return fn(dst, src, et, ev_m, s0f, s1f, Bt, Ct, zagg)


def _comb_body(a0s, a1s, aps, s0s, s1s,
               a0v, a1v, apv, s0v, s1v,
               a0t, a1t, apt, s0t, s1t, coef, out):
    acc = None
    for ci, (a0, a1, ap, s0, s1) in enumerate(
            ((a0s, a1s, aps, s0s, s1s),
             (a0v, a1v, apv, s0v, s1v),
             (a0t, a1t, apt, s0t, s1t))):
        sa = jnp.where(s0[...] + s1[...] > 0, 1.0, 0.0)
        h = _leaky_j(a0[...] + a1[...] + ap[...] * sa)
        term = coef[ci] * h
        acc = term if acc is None else acc + term
    out[...] = acc


def _run_combine(parts, coef):
    f32 = jnp.float32
    nb = lambda i: (i, 0)
    node = pl.BlockSpec((NB, D), nb)
    node1 = pl.BlockSpec((NB, 1), nb)
    spec = [node, node, node, node1, node1] * 3
    return pl.pallas_call(
        _comb_body, grid=(N // NB,),
        in_specs=spec + [pl.BlockSpec(memory_space=pltpu.SMEM)],
        out_specs=node,
        out_shape=jax.ShapeDtypeStruct((N, D), f32),
    )(*parts, coef)


def kernel(x, edge_index, edge_type, visual, textual, struct_emb, rel_emb,
           W1_s, b1_s, W2_s, W1_v, b1_v, W2_v, W1_t, b1_t, W2_t,
           Wv_proj, bv_proj, Wt_proj, bt_proj, alpha, gamma):
    f32 = jnp.float32
    src = edge_index[0]
    dst = edge_index[1]
    et = edge_type

    outs = _run_dense(
        visual, textual, struct_emb, rel_emb,
        Wv_proj, bv_proj.reshape(1, D), Wt_proj, bt_proj.reshape(1, D),
        W1_s, b1_s.reshape(1, D), W2_s,
        W1_v, b1_v.reshape(1, D), W2_v,
        W1_t, b1_t.reshape(1, D), W2_t)
    (ap_s, b_s, p_s, q_s, c_s, r_s,
     ap_v, b_v, p_v, q_v, c_v, r_v,
     ap_t, b_t, p_t, q_t, c_t, r_t, mx) = outs

    pq = jnp.concatenate(
        [p_s.reshape(N), q_s.reshape(N), p_v.reshape(N),
         q_v.reshape(N), p_t.reshape(N), q_t.reshape(N)])
    rtab = jnp.concatenate(
        [r_s.reshape(NREL), r_v.reshape(NREL), r_t.reshape(NREL)])
    mb = mx[0, :3] + mx[1, :3] + mx[2, :3]
    mvals = jnp.where(mb >= 0, mb, 0.01 * mb)
    mvec = jnp.zeros((16,), f32).at[:3].set(mvals)
    zz = jnp.zeros((3 * NPAD,), f32)

    s_out, ev0, ev1, ev2 = _run_passa(dst, src, et, pq, rtab, mvec, zz)
    ev = (ev0, ev1, ev2)

    zagg = jnp.zeros((NPAD, 128), f32)
    parts = []
    for m, (Bt, Ct, Ap) in enumerate(((b_s, c_s, ap_s),
                                      (b_v, c_v, ap_v),
                                      (b_t, c_t, ap_t))):
        s0f = s_out[0, m * NPAD:(m + 1) * NPAD]
        s1f = s_out[1, m * NPAD:(m + 1) * NPAD]
        agg = _run_passb(dst, src, et, ev[m], s0f, s1f, Bt, Ct, zagg)
        parts.extend([agg[0, :N], agg[1, :N], Ap,
                      s0f[:N].reshape(N, 1), s1f[:N].reshape(N, 1)])

    cs = 1.0 - alpha - gamma
    coef = jnp.stack([cs, alpha, gamma, jnp.zeros_like(alpha)]).astype(f32)
    return _run_combine(parts, coef)
